# baseline (speedup 1.0000x reference)
import functools

import jax
import jax.numpy as jnp
from jax import lax
from jax.experimental import pallas as pl
from jax.experimental.pallas import tpu as pltpu
from jax.experimental.pallas import tpu_sc as plsc

BATCH = 16384
NUM_FACTORS = 8
MLP_DIM = 32

_info = plsc.get_sparse_core_info()
_NC, _NS = _info.num_cores, _info.num_subcores
NW = _NC * _NS
B_PER_W = BATCH // NW
CHUNK = 128
NCHUNK = B_PER_W // CHUNK


def _sc_gather_body(users_hbm, items_hbm, ug_t, ig_t, um_t, im_t,
                    ug_o, ig_o, eu_o, ei_o,
                    idx_u, idx_i, ug_v, ig_v, eu_v, ei_v, sem):
    wid = lax.axis_index("s") * _NC + lax.axis_index("c")
    base = wid * B_PER_W
    pltpu.sync_copy(users_hbm.at[wid], idx_u)
    pltpu.sync_copy(items_hbm.at[wid], idx_i)
    copies = []
    for j in range(NCHUNK):
        dst = pl.ds(j * CHUNK, CHUNK)
        copies.append(pltpu.async_copy(ug_t.at[idx_u.at[j]], ug_v.at[dst], sem))
        copies.append(pltpu.async_copy(ig_t.at[idx_i.at[j]], ig_v.at[dst], sem))
        copies.append(pltpu.async_copy(um_t.at[idx_u.at[j]], eu_v.at[dst], sem))
        copies.append(pltpu.async_copy(im_t.at[idx_i.at[j]], ei_v.at[dst], sem))
    for c in copies:
        c.wait()
    out_rows = pl.ds(base, B_PER_W)
    pltpu.sync_copy(ug_v, ug_o.at[out_rows])
    pltpu.sync_copy(ig_v, ig_o.at[out_rows])
    pltpu.sync_copy(eu_v, eu_o.at[out_rows])
    pltpu.sync_copy(ei_v, ei_o.at[out_rows])


_sc_gather = pl.kernel(
    _sc_gather_body,
    out_type=(
        jax.ShapeDtypeStruct((BATCH, NUM_FACTORS), jnp.float32),
        jax.ShapeDtypeStruct((BATCH, NUM_FACTORS), jnp.float32),
        jax.ShapeDtypeStruct((BATCH, MLP_DIM), jnp.float32),
        jax.ShapeDtypeStruct((BATCH, MLP_DIM), jnp.float32),
    ),
    mesh=plsc.VectorSubcoreMesh(core_axis_name="c", subcore_axis_name="s"),
    scratch_types=[
        pltpu.VMEM((NCHUNK, CHUNK), jnp.int32),
        pltpu.VMEM((NCHUNK, CHUNK), jnp.int32),
        pltpu.VMEM((B_PER_W, NUM_FACTORS), jnp.float32),
        pltpu.VMEM((B_PER_W, NUM_FACTORS), jnp.float32),
        pltpu.VMEM((B_PER_W, MLP_DIM), jnp.float32),
        pltpu.VMEM((B_PER_W, MLP_DIM), jnp.float32),
        pltpu.SemaphoreType.DMA,
    ],
    compiler_params=pltpu.CompilerParams(use_tc_tiling_on_sc=False),
    name="neumf_sc_gather",
)


def _tc_mlp_body(ug_ref, ig_ref, eu_ref, ei_ref, w1_ref, b1_ref, w2_ref,
                 b2_ref, w3_ref, b3_ref, wl_ref, bl_ref, out_ref):
    dn = (((1,), (1,)), ((), ()))
    f32 = jnp.float32
    w1 = w1_ref[...]
    h1 = lax.dot_general(eu_ref[...], w1[:, :MLP_DIM], dn, preferred_element_type=f32)
    h1 = h1 + lax.dot_general(ei_ref[...], w1[:, MLP_DIM:], dn, preferred_element_type=f32)
    h1 = jnp.maximum(h1 + b1_ref[...], 0.0)
    h2 = lax.dot_general(h1, w2_ref[...], dn, preferred_element_type=f32)
    h2 = jnp.maximum(h2 + b2_ref[...], 0.0)
    h3 = lax.dot_general(h2, w3_ref[...], dn, preferred_element_type=f32)
    h3 = jnp.maximum(h3 + b3_ref[...], 0.0)
    gmf = ug_ref[...] * ig_ref[...]
    wl = wl_ref[...]
    out = lax.dot_general(gmf, wl[:, :NUM_FACTORS], dn, preferred_element_type=f32)
    out = out + lax.dot_general(h3, wl[:, NUM_FACTORS:], dn, preferred_element_type=f32)
    out_ref[...] = out + bl_ref[...]


def _tc_mlp(ug, ig, eu, ei, W1, b1, W2, b2, W3, b3, Wl, bl):
    bs = 2048
    grid = (BATCH // bs,)
    row = lambda i: (i, 0)
    rep = lambda i: (0, 0)
    return pl.pallas_call(
        _tc_mlp_body,
        grid=grid,
        in_specs=[
            pl.BlockSpec((bs, NUM_FACTORS), row),
            pl.BlockSpec((bs, NUM_FACTORS), row),
            pl.BlockSpec((bs, MLP_DIM), row),
            pl.BlockSpec((bs, MLP_DIM), row),
            pl.BlockSpec(W1.shape, rep),
            pl.BlockSpec((1, 32), rep),
            pl.BlockSpec(W2.shape, rep),
            pl.BlockSpec((1, 16), rep),
            pl.BlockSpec(W3.shape, rep),
            pl.BlockSpec((1, 8), rep),
            pl.BlockSpec((1, 16), rep),
            pl.BlockSpec((1, 1), rep),
        ],
        out_specs=pl.BlockSpec((bs, 1), row),
        out_shape=jax.ShapeDtypeStruct((BATCH, 1), jnp.float32),
        name="neumf_tc_mlp",
    )(ug, ig, eu, ei, W1, b1, W2, b2, W3, b3, Wl, bl)


def kernel(users, items, Ug, Ig, Um, Im, W1, b1, W2, b2, W3, b3, Wl, bl):
    u3 = users.astype(jnp.int32).reshape(NW, NCHUNK, CHUNK)
    i3 = items.astype(jnp.int32).reshape(NW, NCHUNK, CHUNK)
    ug, ig, eu, ei = _sc_gather(u3, i3, Ug, Ig, Um, Im)
    out = _tc_mlp(ug, ig, eu, ei,
                  W1, b1.reshape(1, -1), W2, b2.reshape(1, -1),
                  W3, b3.reshape(1, -1), Wl, bl.reshape(1, 1))
    return out.reshape(-1)
